# baseline (speedup 1.0000x reference)
import functools

import jax
import jax.numpy as jnp
from jax import lax
from jax.experimental import pallas as pl
from jax.experimental.pallas import tpu as pltpu
from jax.experimental.pallas import tpu_sc as plsc

NC, NS = 2, 16
NW = NC * NS
LANE = 16


def _transpose_w(W):
    V, E = W.shape
    WT = jnp.transpose(W)
    nslab = V // 128
    vtail = V - nslab * 128
    wtail = lax.slice(W, (nslab * 128, 0), (V, E)).reshape(-1)
    trips = (nslab + NW - 1) // NW
    mesh = plsc.VectorSubcoreMesh(core_axis_name="c", subcore_axis_name="s")

    @functools.partial(
        pl.kernel,
        out_type=jax.ShapeDtypeStruct((V * E,), jnp.float32),
        mesh=mesh,
        scratch_types=[
            pltpu.VMEM((2, E, 128), jnp.float32),
            pltpu.VMEM((128 * E,), jnp.float32),
            pltpu.VMEM((128 * E,), jnp.float32),
            pltpu.SemaphoreType.DMA,
            pltpu.SemaphoreType.DMA,
            pltpu.SemaphoreType.DMA,
            pltpu.SemaphoreType.DMA,
        ],
        compiler_params=pltpu.CompilerParams(needs_layout_passes=False),
    )
    def tr(wt_hbm, wtail_hbm, wf_hbm, slab, st0, st1, l0, l1, s0, s1):
        wid = lax.axis_index("s") * NC + lax.axis_index("c")
        sts = (st0, st1)
        lsems = (l0, l1)
        ssems = (s0, s1)
        iota = lax.iota(jnp.int32, LANE)
        posv = [iota * E + (LANE * E) * j for j in range(128 // LANE)]

        @pl.when(wid == NW - 1)
        def _tail():
            pltpu.sync_copy(wtail_hbm, wf_hbm.at[pl.ds(nslab * 128 * E, vtail * E)])

        def fetch(k, b):
            u = k * NW + wid

            @pl.when(u < nslab)
            def _():
                for i in (0, 1):
                    @pl.when(b == i)
                    def _(i=i):
                        pltpu.async_copy(
                            wt_hbm.at[:, pl.ds(u * 128, 128)], slab.at[i],
                            lsems[i],
                        )

        def fetch_wait(b):
            for i in (0, 1):
                @pl.when(b == i)
                def _(i=i):
                    pltpu.make_async_copy(
                        wt_hbm.at[:, pl.ds(0, 128)], slab.at[0], lsems[i]
                    ).wait()

        def store_wait(b):
            for i in (0, 1):
                @pl.when(b == i)
                def _(i=i):
                    pltpu.make_async_copy(
                        st0, wf_hbm.at[pl.ds(0, 128 * E)], ssems[i]
                    ).wait()

        fetch(0, 0)

        def body(k, carry):
            b = k % 2
            u = k * NW + wid
            fetch(k + 1, 1 - b)

            @pl.when(u < nslab)
            def _():
                fetch_wait(b)

                @pl.when(k >= 2)
                def _():
                    store_wait(b)

                for i in (0, 1):
                    @pl.when(b == i)
                    def _(i=i):
                        def row(r, c2):
                            for j in range(128 // LANE):
                                val = slab.at[i].at[r][pl.ds(j * LANE, LANE)]
                                plsc.store_scatter(sts[i], [posv[j] + r], val)
                            return c2

                        lax.fori_loop(0, E, row, 0)

                        pltpu.async_copy(
                            sts[i], wf_hbm.at[pl.ds(u * 128 * E, 128 * E)],
                            ssems[i],
                        )

            return carry

        lax.fori_loop(0, trips, body, 0)
        store_wait(0)
        store_wait(1)

    return tr(WT, wtail)


def kernel(x, W):
    B, L = x.shape
    V, E = W.shape
    N = B * L
    BT = B // 128
    NU = L * BT
    per_w = NU // NW
    assert per_w * NW == NU and E % LANE == 0 and per_w >= 3

    wf = _transpose_w(W)
    W2 = wf.reshape(V, E)
    xT = jnp.transpose(x).reshape(NU, 128).astype(jnp.int32)

    mesh = plsc.VectorSubcoreMesh(core_axis_name="c", subcore_axis_name="s")

    @functools.partial(
        pl.kernel,
        out_type=jax.ShapeDtypeStruct((N * E,), jnp.float32),
        mesh=mesh,
        scratch_types=[
            pltpu.VMEM((2, 1, 128), jnp.int32),
            pltpu.VMEM((2, 128, E), jnp.float32),
            pltpu.VMEM((E // 8, 1024), jnp.float32),
            pltpu.VMEM((E // 8, 1024), jnp.float32),
            pltpu.SemaphoreType.DMA,
            pltpu.SemaphoreType.DMA,
            pltpu.SemaphoreType.DMA,
            pltpu.SemaphoreType.DMA,
            pltpu.SemaphoreType.DMA,
            pltpu.SemaphoreType.DMA,
        ],
        compiler_params=pltpu.CompilerParams(
            use_tc_tiling_on_sc=False, needs_layout_passes=False
        ),
    )
    def emb(x_hbm, w_hbm, out_hbm, idxv, gath, st0, st1,
            i0, i1, g0, g1, o0, o1):
        wid = lax.axis_index("s") * NC + lax.axis_index("c")
        u0 = wid * per_w
        sts = (st0, st1)
        isems = (i0, i1)
        gsems = (g0, g1)
        osems = (o0, o1)
        iota = lax.iota(jnp.int32, LANE)
        posv = lax.bitwise_and(iota, 7) * 128
        ethi = lax.shift_right_logical(iota, 3)
        etv = [2 * j + ethi for j in range(E // LANE)]

        def idx_fetch(k, b):
            for i in (0, 1):
                @pl.when(b == i)
                def _(i=i):
                    pltpu.async_copy(
                        x_hbm.at[pl.ds(u0 + k, 1)], idxv.at[i], isems[i]
                    )

        def idx_wait(b):
            for i in (0, 1):
                @pl.when(b == i)
                def _(i=i):
                    pltpu.make_async_copy(
                        x_hbm.at[pl.ds(0, 1)], idxv.at[0], isems[i]
                    ).wait()

        def gather(b):
            for i in (0, 1):
                @pl.when(b == i)
                def _(i=i):
                    pltpu.async_copy(
                        w_hbm.at[idxv.at[i, 0]], gath.at[i], gsems[i]
                    )

        def gather_wait(b):
            for i in (0, 1):
                @pl.when(b == i)
                def _(i=i):
                    pltpu.make_async_copy(
                        w_hbm.at[pl.ds(0, 128)], gath.at[0], gsems[i]
                    ).wait()

        def store(k, b):
            l = (u0 + k) // BT
            bt = (u0 + k) % BT
            for i in (0, 1):
                @pl.when(b == i)
                def _(i=i):
                    for et in range(E // 8):
                        pltpu.async_copy(
                            sts[i].at[et],
                            out_hbm.at[
                                pl.ds(l * (E * B) + et * (8 * B) + bt * 1024,
                                      1024)
                            ],
                            osems[i],
                        )

        def store_wait(b):
            for i in (0, 1):
                @pl.when(b == i)
                def _(i=i):
                    for et in range(E // 8):
                        pltpu.make_async_copy(
                            st0.at[et], out_hbm.at[pl.ds(et * 1024, 1024)],
                            osems[i],
                        ).wait()

        idx_fetch(0, 0)
        idx_wait(0)
        gather(0)
        idx_fetch(1, 1)

        def body(k, carry):
            b = k % 2

            @pl.when(k + 1 < per_w)
            def _():
                idx_wait(1 - b)
                gather(1 - b)

            gather_wait(b)

            @pl.when(k + 2 < per_w)
            def _():
                idx_fetch(k + 2, b)

            @pl.when(k >= 2)
            def _():
                store_wait(b)

            for i in (0, 1):
                @pl.when(b == i)
                def _(i=i):
                    def row(t, c2):
                        pv = posv + t
                        for j in range(E // LANE):
                            val = gath.at[i].at[t][pl.ds(j * LANE, LANE)]
                            plsc.store_scatter(sts[i], [etv[j], pv], val)
                        return c2

                    lax.fori_loop(0, 128, row, 0)

            store(k, b)
            return carry

        lax.fori_loop(0, per_w, body, 0)
        store_wait(per_w % 2)
        store_wait(1 - per_w % 2)

    out1d = emb(xT, W2)
    out5 = out1d.reshape(L, E // 8, BT, 8, 128)
    return out5.transpose(2, 4, 0, 1, 3).reshape(B, L, E)


# speedup vs baseline: 1.7846x; 1.7846x over previous
import functools

import jax
import jax.numpy as jnp
from jax import lax
from jax.experimental import pallas as pl
from jax.experimental.pallas import tpu as pltpu
from jax.experimental.pallas import tpu_sc as plsc

NC, NS = 2, 16
NW = NC * NS
IDX_MINOR = 128
GPC = 5
CHUNK = IDX_MINOR * GPC


def kernel(x, W):
    B, L = x.shape
    V, E = W.shape
    N = B * L
    assert N % (NW * IDX_MINOR) == 0
    per_w = N // NW
    rows_per_w = per_w // IDX_MINOR
    nchunks = per_w // CHUNK
    assert nchunks * CHUNK == per_w and nchunks % 2 == 0
    npairs = nchunks // 2

    x2d = x.reshape(N // IDX_MINOR, IDX_MINOR).astype(jnp.int32)

    mesh = plsc.VectorSubcoreMesh(core_axis_name="c", subcore_axis_name="s")

    @functools.partial(
        pl.kernel,
        out_type=jax.ShapeDtypeStruct((N, E), jnp.float32),
        mesh=mesh,
        scratch_types=[
            pltpu.VMEM((rows_per_w, IDX_MINOR), jnp.int32),
            pltpu.VMEM((CHUNK, E), jnp.float32),
            pltpu.VMEM((CHUNK, E), jnp.float32),
            pltpu.SemaphoreType.DMA,
            pltpu.SemaphoreType.DMA,
            pltpu.SemaphoreType.DMA,
            pltpu.SemaphoreType.DMA,
        ],
        compiler_params=pltpu.CompilerParams(
            use_tc_tiling_on_sc=False, skip_device_barrier=True
        ),
    )
    def emb(x_hbm, w_hbm, out_hbm, idx_v, buf0, buf1, gsem0, gsem1, ssem0,
            ssem1):
        wid = lax.axis_index("s") * NC + lax.axis_index("c")
        row0 = wid * rows_per_w
        base = wid * per_w
        pltpu.sync_copy(x_hbm.at[pl.ds(row0, rows_per_w)], idx_v)

        bufs = (buf0, buf1)
        gsems = (gsem0, gsem1)
        ssems = (ssem0, ssem1)

        def issue_gathers(g, b):
            for j in range(GPC):
                pltpu.async_copy(
                    w_hbm.at[idx_v.at[g * GPC + j]],
                    bufs[b].at[pl.ds(j * IDX_MINOR, IDX_MINOR)],
                    gsems[b],
                )

        def wait_gathers(b):
            pltpu.make_async_copy(
                out_hbm.at[pl.ds(base, CHUNK)], bufs[b], gsems[b]
            ).wait()

        def issue_store(g, b):
            pltpu.async_copy(
                bufs[b], out_hbm.at[pl.ds(base + g * CHUNK, CHUNK)], ssems[b]
            )

        def wait_store(b):
            pltpu.make_async_copy(
                bufs[b], out_hbm.at[pl.ds(base, CHUNK)], ssems[b]
            ).wait()

        issue_gathers(0, 0)
        issue_gathers(1, 1)

        def body(i, carry):
            g0 = 2 * i
            wait_gathers(0)
            issue_store(g0, 0)
            wait_gathers(1)
            issue_store(g0 + 1, 1)
            wait_store(0)
            issue_gathers(g0 + 2, 0)
            wait_store(1)
            issue_gathers(g0 + 3, 1)
            return carry

        lax.fori_loop(0, npairs - 1, body, 0)

        g0 = nchunks - 2
        wait_gathers(0)
        issue_store(g0, 0)
        wait_gathers(1)
        issue_store(g0 + 1, 1)
        wait_store(0)
        wait_store(1)

    out = emb(x2d, W)
    return out.reshape(B, L, E)
